# Initial kernel scaffold; baseline (speedup 1.0000x reference)
import jax
import jax.numpy as jnp
from jax.experimental import pallas as pl


def kernel(input1, input2):
    raise NotImplementedError("write your pallas kernel here")


# speedup vs baseline: 1.2243x; 1.2243x over previous
import dataclasses
import functools

import jax
import jax.numpy as jnp
from jax import lax
from jax.experimental import pallas as pl
from jax.experimental.pallas import tpu as pltpu
from jax.experimental.pallas import tpu_sc as plsc

_NC = 2
_NS = 16
_NW = _NC * _NS
_P = 64


def _idx_w_body(H, W, f_ref, idx_ref, w_ref):
    b = pl.program_id(0)
    fx = f_ref[0, 0]
    fy = f_ref[0, 1]
    col = lax.broadcasted_iota(jnp.int32, (H, W), 1).astype(jnp.float32)
    row = lax.broadcasted_iota(jnp.int32, (H, W), 0).astype(jnp.float32)
    xn = 2.0 * (col + fx) / (W - 1.0) - 1.0
    yn = 2.0 * (row + fy) / (H - 1.0) - 1.0
    x = (xn + 1.0) * (W - 1.0) / 2.0
    y = (yn + 1.0) * (H - 1.0) / 2.0
    xb = jnp.clip(jnp.floor(x), 0.0, W - 2.0)
    yb = jnp.clip(jnp.floor(y), 0.0, H - 2.0)
    wx0 = jnp.maximum(0.0, 1.0 - jnp.abs(x - xb))
    wx1 = jnp.maximum(0.0, 1.0 - jnp.abs(x - (xb + 1.0)))
    wy0 = jnp.maximum(0.0, 1.0 - jnp.abs(y - yb))
    wy1 = jnp.maximum(0.0, 1.0 - jnp.abs(y - (yb + 1.0)))
    base = b * (H * W) + yb.astype(jnp.int32) * W + xb.astype(jnp.int32)
    idx_ref[0, 0] = base
    idx_ref[1, 0] = base + 1
    idx_ref[2, 0] = base + W
    idx_ref[3, 0] = base + W + 1
    w_ref[0, 0] = wy0 * wx0
    w_ref[1, 0] = wy0 * wx1
    w_ref[2, 0] = wy1 * wx0
    w_ref[3, 0] = wy1 * wx1


def _idx_w(input2):
    B, _, H, W = input2.shape
    return pl.pallas_call(
        functools.partial(_idx_w_body, H, W),
        grid=(B,),
        in_specs=[pl.BlockSpec((1, 2, H, W), lambda b: (b, 0, 0, 0))],
        out_specs=[
            pl.BlockSpec((4, 1, H, W), lambda b: (0, b, 0, 0)),
            pl.BlockSpec((4, 1, H, W), lambda b: (0, b, 0, 0)),
        ],
        out_shape=[
            jax.ShapeDtypeStruct((4, B, H, W), jnp.int32),
            jax.ShapeDtypeStruct((4, B, H, W), jnp.float32),
        ],
    )(input2)


def _sc_warp(table, idxc, wc):
    N, C = table.shape
    PW = N // _NW
    n_chunks = PW // _P
    CP4 = 4 * _P
    mesh = plsc.VectorSubcoreMesh(core_axis_name="c", subcore_axis_name="s")
    cp = pltpu.CompilerParams()
    if "needs_layout_passes" in pltpu.CompilerParams.__dataclass_fields__:
        cp = dataclasses.replace(cp, needs_layout_passes=False)
    if "use_tc_tiling_on_sc" in pltpu.CompilerParams.__dataclass_fields__:
        cp = dataclasses.replace(cp, use_tc_tiling_on_sc=False)

    @functools.partial(
        pl.kernel,
        mesh=mesh,
        compiler_params=cp,
        out_type=jax.ShapeDtypeStruct((N, C), jnp.bfloat16),
        scratch_types=[
            pltpu.VMEM((3, CP4), jnp.int32),
            pltpu.VMEM((3, CP4), jnp.float32),
            pltpu.VMEM((CP4, C), jnp.bfloat16),
            pltpu.VMEM((CP4, C), jnp.bfloat16),
            pltpu.VMEM((CP4, C), jnp.bfloat16),
            pltpu.VMEM((_P, C), jnp.bfloat16),
            pltpu.VMEM((_P, C), jnp.bfloat16),
            pltpu.VMEM((_P, C), jnp.bfloat16),
        ] + [pltpu.SemaphoreType.DMA] * 9,
    )
    def warp(table_hbm, idx_hbm, w_hbm, out_hbm,
             idx_v, w_v, g0, g1, g2, o0, o1, o2,
             sl0, sl1, sl2, sg0, sg1, sg2, so0, so1, so2):
        g = (g0, g1, g2)
        o = (o0, o1, o2)
        sl = (sl0, sl1, sl2)
        sg = (sg0, sg1, sg2)
        so = (so0, so1, so2)
        wid = lax.axis_index("c") * _NS + lax.axis_index("s")
        c0 = wid * n_chunks

        def fire_load(t, s):
            off = (c0 + t) * CP4
            pltpu.async_copy(idx_hbm.at[pl.ds(off, CP4)], idx_v.at[s], sl[s])
            pltpu.async_copy(w_hbm.at[pl.ds(off, CP4)], w_v.at[s], sl[s])

        def drain_load(s):
            pltpu.make_async_copy(idx_hbm.at[pl.ds(0, CP4)], idx_v.at[s],
                                  sl[s]).wait()
            pltpu.make_async_copy(w_hbm.at[pl.ds(0, CP4)], w_v.at[s],
                                  sl[s]).wait()

        def fire_gather(s):
            for h in range(2):
                pltpu.async_copy(
                    table_hbm.at[idx_v.at[s, pl.ds(h * 2 * _P, 2 * _P)]],
                    g[s].at[pl.ds(h * 2 * _P, 2 * _P)], sg[s])

        def drain_gather(s):
            for h in range(2):
                pltpu.make_async_copy(
                    table_hbm.at[idx_v.at[s, pl.ds(h * 2 * _P, 2 * _P)]],
                    g[s].at[pl.ds(h * 2 * _P, 2 * _P)], sg[s]).wait()

        def fire_out(t, s):
            pltpu.async_copy(o[s], out_hbm.at[pl.ds((c0 + t) * _P, _P)],
                             so[s])

        def drain_out(s):
            pltpu.make_async_copy(o[s], out_hbm.at[pl.ds(0, _P)],
                                  so[s]).wait()

        def blend(s):
            gs = g[s]
            os_ = o[s]

            @pl.loop(0, _P)
            def _pixel(p):
                pw = []
                for k in range(4):
                    wk = plsc.load_gather(
                        w_v, [jnp.full((16,), s, jnp.int32),
                              jnp.full((16,), k * _P, jnp.int32)
                              + jnp.full((16,), p, jnp.int32)])
                    pw.append(plsc.pack(wk, wk,
                                        format=plsc.PackFormat.INTERLEAVED))
                for r in range(C // 32):
                    slc = pl.ds(r * 32, 32)
                    os_[p, slc] = (gs[p, slc] * pw[0]
                                   + gs[_P + p, slc] * pw[1]
                                   + gs[2 * _P + p, slc] * pw[2]
                                   + gs[3 * _P + p, slc] * pw[3])

        fire_load(0, 0)
        fire_load(1, 1)
        fire_load(2, 2)
        drain_load(0)
        fire_gather(0)
        drain_load(1)
        fire_gather(1)

        @pl.loop(0, n_chunks, step=3)
        def _body(ci):
            for j in range(3):
                s = j
                t = ci + j
                drain_gather(s)

                @pl.when(t + 2 < n_chunks)
                def _prefetch_gather():
                    drain_load((s + 2) % 3)
                    fire_gather((s + 2) % 3)

                @pl.when(t >= 3)
                def _recycle_out():
                    drain_out(s)

                blend(s)
                fire_out(t, s)

                @pl.when(t + 3 < n_chunks)
                def _prefetch_load():
                    fire_load(t + 3, s)

        drain_out(0)
        drain_out(1)
        drain_out(2)

    return warp(table, idxc, wc)


def kernel(input1, input2):
    B, C, H, W = input1.shape
    N = B * H * W
    table = jnp.transpose(input1, (0, 2, 3, 1)).reshape(N, C).astype(jnp.bfloat16)
    idx4, w4 = _idx_w(input2)
    idxc = idx4.reshape(4, N // _P, _P).transpose(1, 0, 2).reshape(4 * N)
    wc = w4.reshape(4, N // _P, _P).transpose(1, 0, 2).reshape(4 * N)
    out_t = _sc_warp(table, idxc, wc)
    return jnp.transpose(out_t.reshape(B, H, W, C).astype(jnp.float32),
                         (0, 3, 1, 2))
